# Initial kernel scaffold; baseline (speedup 1.0000x reference)
import jax
import jax.numpy as jnp
from jax.experimental import pallas as pl


def kernel(id_, emb_weight):
    raise NotImplementedError("write your pallas kernel here")


# speedup vs baseline: 6.5409x; 6.5409x over previous
import math

import jax
import jax.numpy as jnp
from jax import lax
from jax.experimental import pallas as pl
from jax.experimental.pallas import tpu as pltpu
from jax.experimental.pallas import tpu_sc as plsc

_NUM_EMB = 1_000_000
_DIM = 32
_K = 8
_ROWS = 16384
_COLS = 26
_N = _ROWS * _COLS
_NC = 2
_NS = 16
_NW = _NC * _NS
_PER_W = _N // _NW
_E = 128
_UNITS = _PER_W // _E
_LANES = 16
_INV_MOD = 1.0 / _NUM_EMB
_SCALE = 1.0 / math.sqrt(_K)


def _shifted_index(x, j):
    xs = x << j
    q = (xs.astype(jnp.float32) * _INV_MOD).astype(jnp.int32)
    r = xs - q * _NUM_EMB
    r = jnp.where(r < 0, r + _NUM_EMB, r)
    r = jnp.where(r >= _NUM_EMB, r - _NUM_EMB, r)
    return r


def _body(table_hbm, ids_hbm, out_hbm, ids_v, idx_vs, acc_v, sem):
    wid = lax.axis_index("s") * _NC + lax.axis_index("c")
    zero = jnp.zeros((_LANES,), jnp.float32)

    def unit(u, carry):
        base = wid * jnp.int32(_PER_W) + u * jnp.int32(_E)
        pltpu.sync_copy(ids_hbm.at[pl.ds(base, _E)], ids_v)
        for i in range(_E // _LANES):
            x = ids_v[pl.ds(i * _LANES, _LANES)]
            for j in range(_K):
                idx_vs[j][pl.ds(i * _LANES, _LANES)] = _shifted_index(x, j)
        for row in range(_E):
            acc_v[row, pl.ds(0, _LANES)] = zero
            acc_v[row, pl.ds(_LANES, _LANES)] = zero
        cps = [pltpu.async_copy(table_hbm.at[idx_vs[j]], acc_v, sem, add=True)
               for j in range(_K)]
        for cp in cps:
            cp.wait()
        for row in range(_E):
            for h in (0, _LANES):
                acc_v[row, pl.ds(h, _LANES)] = acc_v[row, pl.ds(h, _LANES)] * _SCALE
        pltpu.sync_copy(acc_v, out_hbm.at[pl.ds(base, _E)])
        return carry

    lax.fori_loop(jnp.int32(0), jnp.int32(_UNITS), unit, jnp.int32(0))


def kernel(id_, emb_weight):
    ids = id_.reshape(_N).astype(jnp.int32)
    mesh = plsc.VectorSubcoreMesh(
        core_axis_name="c", subcore_axis_name="s",
        num_cores=_NC, num_subcores=_NS)
    out = pl.kernel(
        _body,
        out_type=jax.ShapeDtypeStruct((_N, _DIM), jnp.float32),
        mesh=mesh,
        compiler_params=pltpu.CompilerParams(use_tc_tiling_on_sc=False),
        scratch_types=[
            pltpu.VMEM((_E,), jnp.int32),
            [pltpu.VMEM((_E,), jnp.int32) for _ in range(_K)],
            pltpu.VMEM((_E, _DIM), jnp.float32),
            pltpu.SemaphoreType.DMA,
        ],
    )(emb_weight, ids)
    return out.reshape(_ROWS, _COLS, _DIM)


# speedup vs baseline: 7.6601x; 1.1711x over previous
import math

import jax
import jax.numpy as jnp
from jax import lax
from jax.experimental import pallas as pl
from jax.experimental.pallas import tpu as pltpu
from jax.experimental.pallas import tpu_sc as plsc

_NUM_EMB = 1_000_000
_HALF = _NUM_EMB // 2
_DIM = 32
_K = 8
_ROWS = 16384
_COLS = 26
_N = _ROWS * _COLS
_NC = 2
_NS = 16
_NW = _NC * _NS
_PER_W = _N // _NW
_E = 128
_UNITS = _PER_W // _E
_LANES = 16
_NBUF = 4
_SCALE = 1.0 / math.sqrt(_K)


def _body(table_hbm, ids_hbm, out_hbm, ids_all, idx_vs, acc_vs, sem_g, sem_o):
    wid = lax.axis_index("s") * _NC + lax.axis_index("c")
    wbase = wid * jnp.int32(_PER_W)
    pltpu.sync_copy(ids_hbm.at[pl.ds(wbase, _PER_W)], ids_all)
    zero = jnp.zeros((_LANES,), jnp.float32)

    def compute_idx(u, b):
        off = u * jnp.int32(_E)

        def grp(i, c):
            i16 = i * jnp.int32(_LANES)
            x = ids_all[pl.ds(off + i16, _LANES)]
            idx_vs[b][0][pl.ds(i16, _LANES)] = x
            r = x
            for j in range(1, _K):
                r2 = r + r
                r = jnp.where(r >= _HALF, r2 - _NUM_EMB, r2)
                idx_vs[b][j][pl.ds(i16, _LANES)] = r
            return c

        lax.fori_loop(jnp.int32(0), jnp.int32(_E // _LANES), grp, jnp.int32(0))

    def zero_acc(b):
        def z(i, c):
            row = i * jnp.int32(8)
            for rr in range(8):
                acc_vs[b][row + rr, pl.ds(0, _LANES)] = zero
                acc_vs[b][row + rr, pl.ds(_LANES, _LANES)] = zero
            return c

        lax.fori_loop(jnp.int32(0), jnp.int32(_E // 8), z, jnp.int32(0))

    def scale_acc(b):
        def s(i, c):
            row = i * jnp.int32(8)
            for rr in range(8):
                for h in (0, _LANES):
                    v = acc_vs[b][row + rr, pl.ds(h, _LANES)]
                    acc_vs[b][row + rr, pl.ds(h, _LANES)] = v * _SCALE
            return c

        lax.fori_loop(jnp.int32(0), jnp.int32(_E // 8), s, jnp.int32(0))

    def wait_out(b):
        pltpu.make_async_copy(
            acc_vs[b], out_hbm.at[pl.ds(0, _E)], sem_o[b]).wait()

    def wait_gathers(b):
        for j in range(_K):
            pltpu.make_async_copy(
                table_hbm.at[idx_vs[b][j]], acc_vs[b], sem_g[b]).wait()

    def issue(u, b, wait_mode):
        compute_idx(u, b)
        if wait_mode == "always":
            wait_out(b)
        elif wait_mode == "guard":
            @pl.when(u >= jnp.int32(_NBUF))
            def _():
                wait_out(b)
        zero_acc(b)
        for j in range(_K):
            pltpu.async_copy(
                table_hbm.at[idx_vs[b][j]], acc_vs[b], sem_g[b], add=True)

    def complete(u, b):
        wait_gathers(b)
        scale_acc(b)
        pltpu.async_copy(
            acc_vs[b], out_hbm.at[pl.ds(wbase + u * jnp.int32(_E), _E)],
            sem_o[b])

    for u0 in range(_NBUF - 1):
        issue(jnp.int32(u0), u0, "never")

    def grp(g, carry):
        for b4 in range(_NBUF):
            u_i = jnp.int32(_NBUF - 1) + g * jnp.int32(_NBUF) + jnp.int32(b4)
            issue(u_i, (_NBUF - 1 + b4) % _NBUF, "guard")
            u_c = g * jnp.int32(_NBUF) + jnp.int32(b4)
            complete(u_c, b4)
        return carry

    n_grps = (_UNITS - (_NBUF - 1)) // _NBUF
    lax.fori_loop(jnp.int32(0), jnp.int32(n_grps), grp, jnp.int32(0))

    last = _UNITS - 1
    issue(jnp.int32(last), last % _NBUF, "always")
    for uc in range(_UNITS - _NBUF, _UNITS):
        complete(jnp.int32(uc), uc % _NBUF)
    for b in range(_NBUF):
        wait_out(b)


def kernel(id_, emb_weight):
    ids = id_.reshape(_N).astype(jnp.int32)
    mesh = plsc.VectorSubcoreMesh(
        core_axis_name="c", subcore_axis_name="s",
        num_cores=_NC, num_subcores=_NS)
    out = pl.kernel(
        _body,
        out_type=jax.ShapeDtypeStruct((_N, _DIM), jnp.float32),
        mesh=mesh,
        compiler_params=pltpu.CompilerParams(use_tc_tiling_on_sc=False),
        scratch_types=[
            pltpu.VMEM((_PER_W,), jnp.int32),
            [[pltpu.VMEM((_E,), jnp.int32) for _ in range(_K)]
             for _ in range(_NBUF)],
            [pltpu.VMEM((_E, _DIM), jnp.float32) for _ in range(_NBUF)],
            [pltpu.SemaphoreType.DMA for _ in range(_NBUF)],
            [pltpu.SemaphoreType.DMA for _ in range(_NBUF)],
        ],
    )(emb_weight, ids)
    return out.reshape(_ROWS, _COLS, _DIM)
